# baseline (speedup 1.0000x reference)
import functools

import jax
import jax.numpy as jnp
from jax import lax
from jax.experimental import pallas as pl
from jax.experimental.pallas import tpu as pltpu
from jax.experimental.pallas import tpu_sc as plsc

T = 2048
D = 768
E = 8
F = 3072
CAP = 640
NB = E * CAP
NBP = NB + 8
CH = 128
NC, NS = 2, 16
NW = NC * NS
TPW = T // NW
FC = 3072


def _router_body(x_ref, wr_ref,
                 i0_ref, i1_ref, g0_ref, g1_ref,
                 d0_ref, d1_ref, s0_ref, s1_ref, xb_ref):
    x = x_ref[...]
    wr = wr_ref[...]
    xb_ref[...] = x.astype(jnp.bfloat16)
    logits = jnp.dot(x, wr, preferred_element_type=jnp.float32)
    eidx = lax.broadcasted_iota(jnp.int32, (T, E), 1)
    m0 = jnp.max(logits, axis=1, keepdims=True)
    i0 = jnp.min(jnp.where(logits == m0, eidx, E), axis=1, keepdims=True)
    lmask = jnp.where(eidx == i0, -jnp.inf, logits)
    m1 = jnp.max(lmask, axis=1, keepdims=True)
    i1 = jnp.min(jnp.where(lmask == m1, eidx, E), axis=1, keepdims=True)
    g0 = 1.0 / (1.0 + jnp.exp(m1 - m0))
    g1 = 1.0 - g0

    onehot = ((eidx == i0) | (eidx == i1)).astype(jnp.float32)
    r = lax.broadcasted_iota(jnp.int32, (CH, CH), 0)
    c = lax.broadcasted_iota(jnp.int32, (CH, CH), 1)
    tril = (r >= c).astype(jnp.float32)
    eidx_c = lax.broadcasted_iota(jnp.int32, (CH, E), 1)
    carry = jnp.zeros((1, E), jnp.float32)
    for cb in range(T // CH):
        lo = cb * CH
        mc = lax.slice(onehot, (lo, 0), (lo + CH, E))
        inc = jnp.dot(tril, mc, preferred_element_type=jnp.float32)
        excl = inc - mc + carry
        i0c = lax.slice(i0, (lo, 0), (lo + CH, 1))
        i1c = lax.slice(i1, (lo, 0), (lo + CH, 1))
        g0c = lax.slice(g0, (lo, 0), (lo + CH, 1))
        g1c = lax.slice(g1, (lo, 0), (lo + CH, 1))
        p0 = jnp.sum(jnp.where(eidx_c == i0c, excl, 0.0), axis=1,
                     keepdims=True).astype(jnp.int32)
        p1 = jnp.sum(jnp.where(eidx_c == i1c, excl, 0.0), axis=1,
                     keepdims=True).astype(jnp.int32)
        k0 = p0 < CAP
        k1 = p1 < CAP
        slot0 = i0c * CAP + p0
        slot1 = i1c * CAP + p1
        sl = pl.ds(lo, CH)
        i0_ref[sl, :] = i0c
        i1_ref[sl, :] = i1c
        g0_ref[sl, :] = jnp.broadcast_to(jnp.where(k0, g0c, 0.0), (CH, 16))
        g1_ref[sl, :] = jnp.broadcast_to(jnp.where(k1, g1c, 0.0), (CH, 16))
        d0_ref[sl, :] = jnp.where(k0, slot0, NB)
        d1_ref[sl, :] = jnp.where(k1, slot1, NB)
        sdrop = lax.slice(i0, (0, 0), (1, 1)) * CAP
        s0_ref[sl, :] = jnp.where(k0, slot0, sdrop)
        s1_ref[sl, :] = jnp.where(k1, slot1, sdrop)
        carry = carry + lax.slice(inc, (CH - 1, 0), (CH, E))


_router = pl.pallas_call(
    _router_body,
    out_shape=[
        jax.ShapeDtypeStruct((T, 1), jnp.int32),
        jax.ShapeDtypeStruct((T, 1), jnp.int32),
        jax.ShapeDtypeStruct((T, 16), jnp.float32),
        jax.ShapeDtypeStruct((T, 16), jnp.float32),
        jax.ShapeDtypeStruct((T, 1), jnp.int32),
        jax.ShapeDtypeStruct((T, 1), jnp.int32),
        jax.ShapeDtypeStruct((T, 1), jnp.int32),
        jax.ShapeDtypeStruct((T, 1), jnp.int32),
        jax.ShapeDtypeStruct((T, D), jnp.bfloat16),
    ],
)


@functools.cache
def _sc_mesh():
    return plsc.VectorSubcoreMesh(
        core_axis_name="c", subcore_axis_name="s",
        num_cores=NC, num_subcores=NS)


_H = TPW // 2


_DI = D // 2


@functools.cache
def _get_dispatch():
    @functools.partial(
        pl.kernel,
        out_type=jax.ShapeDtypeStruct((NBP, _DI), jnp.int32),
        mesh=_sc_mesh(),
        scratch_types=[
            pltpu.VMEM((_H,), jnp.int32),
            pltpu.VMEM((_H,), jnp.int32),
            pltpu.VMEM((_H,), jnp.int32),
            pltpu.VMEM((_H,), jnp.int32),
            pltpu.VMEM((_H, _DI), jnp.int32),
            pltpu.VMEM((_H, _DI), jnp.int32),
            pltpu.SemaphoreType.DMA,
            pltpu.SemaphoreType.DMA,
            pltpu.SemaphoreType.DMA,
            pltpu.SemaphoreType.DMA,
        ],
    )
    def _dispatch(x_hbm, d0_hbm, d1_hbm, buf_hbm,
                  i0a_v, i1a_v, i0b_v, i1b_v, ra_v, rb_v,
                  semA, semB, sem0, sem1):
        wid = lax.axis_index("s") * NC + lax.axis_index("c")
        base = wid * TPW
        pltpu.sync_copy(x_hbm.at[pl.ds(base, _H)], ra_v)
        pltpu.sync_copy(x_hbm.at[pl.ds(base + _H, _H)], rb_v)
        pltpu.sync_copy(d0_hbm.at[pl.ds(base, _H)], i0a_v)
        pltpu.sync_copy(d1_hbm.at[pl.ds(base, _H)], i1a_v)
        pltpu.sync_copy(d0_hbm.at[pl.ds(base + _H, _H)], i0b_v)
        pltpu.sync_copy(d1_hbm.at[pl.ds(base + _H, _H)], i1b_v)
        s0a = pltpu.async_copy(ra_v, buf_hbm.at[i0a_v], sem0)
        s1a = pltpu.async_copy(ra_v, buf_hbm.at[i1a_v], sem1)
        s0b = pltpu.async_copy(rb_v, buf_hbm.at[i0b_v], semA)
        s1b = pltpu.async_copy(rb_v, buf_hbm.at[i1b_v], semB)
        s0a.wait()
        s1a.wait()
        s0b.wait()
        s1b.wait()

    return _dispatch


def _ffn_body(b_ref, w1_ref, w2_ref, y_ref):
    f = pl.program_id(1)
    w1b = w1_ref[0].astype(jnp.bfloat16)
    h = jnp.maximum(
        jnp.dot(b_ref[...], w1b, preferred_element_type=jnp.float32), 0.0)
    w2b = w2_ref[0].astype(jnp.bfloat16)
    yc = jnp.dot(h.astype(jnp.bfloat16), w2b,
                 preferred_element_type=jnp.float32)

    @pl.when(f == 0)
    def _init():
        y_ref[...] = yc

    @pl.when(f > 0)
    def _acc():
        y_ref[...] += yc


_ffn = pl.pallas_call(
    _ffn_body,
    grid=(E, F // FC),
    in_specs=[
        pl.BlockSpec((CAP, D), lambda e, f: (e, 0)),
        pl.BlockSpec((1, D, FC), lambda e, f: (e, 0, f)),
        pl.BlockSpec((1, FC, D), lambda e, f: (e, f, 0)),
    ],
    out_specs=pl.BlockSpec((CAP, D), lambda e, f: (e, 0)),
    out_shape=jax.ShapeDtypeStruct((NB, D), jnp.float32),
    compiler_params=pltpu.CompilerParams(
        dimension_semantics=("parallel", "arbitrary")),
)


@functools.cache
def _get_combine():
    @functools.partial(
        pl.kernel,
        out_type=jax.ShapeDtypeStruct((T, D), jnp.float32),
        mesh=_sc_mesh(),
        scratch_types=[
            pltpu.VMEM((_H,), jnp.int32),
            pltpu.VMEM((_H,), jnp.int32),
            pltpu.VMEM((_H,), jnp.int32),
            pltpu.VMEM((_H,), jnp.int32),
            pltpu.VMEM((TPW, 16), jnp.float32),
            pltpu.VMEM((TPW, 16), jnp.float32),
            pltpu.VMEM((_H, D), jnp.float32),
            pltpu.VMEM((_H, D), jnp.float32),
            pltpu.VMEM((_H, D), jnp.float32),
            pltpu.VMEM((_H, D), jnp.float32),
            pltpu.SemaphoreType.DMA,
            pltpu.SemaphoreType.DMA,
            pltpu.SemaphoreType.DMA,
        ],
    )
    def _combine(y_hbm, s0_hbm, s1_hbm, g0_hbm, g1_hbm, o_hbm,
                 i0a_v, i1a_v, i0b_v, i1b_v, g0_v, g1_v,
                 r0a_v, r1a_v, r0b_v, r1b_v, semA, semB, semW):
        wid = lax.axis_index("s") * NC + lax.axis_index("c")
        base = wid * TPW
        pltpu.sync_copy(s0_hbm.at[pl.ds(base, _H)], i0a_v)
        pltpu.sync_copy(s1_hbm.at[pl.ds(base, _H)], i1a_v)
        pltpu.sync_copy(s0_hbm.at[pl.ds(base + _H, _H)], i0b_v)
        pltpu.sync_copy(s1_hbm.at[pl.ds(base + _H, _H)], i1b_v)
        ga0 = pltpu.async_copy(y_hbm.at[i0a_v], r0a_v, semA)
        ga1 = pltpu.async_copy(y_hbm.at[i1a_v], r1a_v, semA)
        gb0 = pltpu.async_copy(y_hbm.at[i0b_v], r0b_v, semB)
        gb1 = pltpu.async_copy(y_hbm.at[i1b_v], r1b_v, semB)
        pltpu.sync_copy(g0_hbm.at[pl.ds(base, TPW)], g0_v)
        pltpu.sync_copy(g1_hbm.at[pl.ds(base, TPW)], g1_v)

        def make_body(r0, r1, off):
            def body(i, carry):
                gv0 = g0_v[i + off, :]
                gv1 = g1_v[i + off, :]
                for j in range(D // 16):
                    sl = pl.ds(j * 16, 16)
                    r0[i, sl] = r0[i, sl] * gv0 + r1[i, sl] * gv1
                return carry
            return body

        ga0.wait()
        ga1.wait()
        lax.fori_loop(0, _H, make_body(r0a_v, r1a_v, 0), 0)
        wa = pltpu.async_copy(r0a_v, o_hbm.at[pl.ds(base, _H)], semW)
        gb0.wait()
        gb1.wait()
        lax.fori_loop(0, _H, make_body(r0b_v, r1b_v, _H), 0)
        wb = pltpu.async_copy(r0b_v, o_hbm.at[pl.ds(base + _H, _H)], semW)
        wa.wait()
        wb.wait()

    return _combine


def kernel(input_tensor, Wr, W1, W2):
    x = input_tensor
    i0, i1, g0, g1, d0, d1, s0, s1, xb = _router(x, Wr)
    xbi = lax.bitcast_convert_type(xb.reshape(T, _DI, 2), jnp.int32)
    bufi = _get_dispatch()(xbi, d0.reshape(T), d1.reshape(T))
    buf = lax.bitcast_convert_type(bufi, jnp.bfloat16).reshape(NBP, D)
    y = _ffn(buf, W1, W2)
    out = _get_combine()(y, s0.reshape(T), s1.reshape(T), g0, g1)
    topi = jnp.concatenate([i0, i1], axis=1)
    return out, topi


# speedup vs baseline: 2.0182x; 2.0182x over previous
import functools

import jax
import jax.numpy as jnp
from jax import lax
from jax.experimental import pallas as pl
from jax.experimental.pallas import tpu as pltpu
from jax.experimental.pallas import tpu_sc as plsc

T = 2048
D = 768
E = 8
F = 3072
CAP = 640
NB = E * CAP
NBP = NB + 8
CH = 128
NC, NS = 2, 16
NW = NC * NS
TPW = T // NW
FC = 3072


def _router_body(x_ref, wr_ref,
                 i0_ref, i1_ref, g0_ref, g1_ref,
                 d0_ref, d1_ref, s0_ref, s1_ref):
    x = x_ref[...]
    wr = wr_ref[...]
    logits = jnp.dot(x, wr, preferred_element_type=jnp.float32)
    eidx = lax.broadcasted_iota(jnp.int32, (T, E), 1)
    m0 = jnp.max(logits, axis=1, keepdims=True)
    i0 = jnp.min(jnp.where(logits == m0, eidx, E), axis=1, keepdims=True)
    lmask = jnp.where(eidx == i0, -jnp.inf, logits)
    m1 = jnp.max(lmask, axis=1, keepdims=True)
    i1 = jnp.min(jnp.where(lmask == m1, eidx, E), axis=1, keepdims=True)
    g0 = 1.0 / (1.0 + jnp.exp(m1 - m0))
    g1 = 1.0 - g0

    onehot = ((eidx == i0) | (eidx == i1)).astype(jnp.float32)
    r = lax.broadcasted_iota(jnp.int32, (CH, CH), 0)
    c = lax.broadcasted_iota(jnp.int32, (CH, CH), 1)
    tril = (r >= c).astype(jnp.float32)
    eidx_c = lax.broadcasted_iota(jnp.int32, (CH, E), 1)
    carry = jnp.zeros((1, E), jnp.float32)
    for cb in range(T // CH):
        lo = cb * CH
        mc = lax.slice(onehot, (lo, 0), (lo + CH, E))
        inc = jnp.dot(tril, mc, preferred_element_type=jnp.float32)
        excl = inc - mc + carry
        i0c = lax.slice(i0, (lo, 0), (lo + CH, 1))
        i1c = lax.slice(i1, (lo, 0), (lo + CH, 1))
        g0c = lax.slice(g0, (lo, 0), (lo + CH, 1))
        g1c = lax.slice(g1, (lo, 0), (lo + CH, 1))
        p0 = jnp.sum(jnp.where(eidx_c == i0c, excl, 0.0), axis=1,
                     keepdims=True).astype(jnp.int32)
        p1 = jnp.sum(jnp.where(eidx_c == i1c, excl, 0.0), axis=1,
                     keepdims=True).astype(jnp.int32)
        k0 = p0 < CAP
        k1 = p1 < CAP
        slot0 = i0c * CAP + p0
        slot1 = i1c * CAP + p1
        sl = pl.ds(lo, CH)
        i0_ref[sl, :] = i0c
        i1_ref[sl, :] = i1c
        g0_ref[sl, :] = jnp.broadcast_to(jnp.where(k0, g0c, 0.0), (CH, 16))
        g1_ref[sl, :] = jnp.broadcast_to(jnp.where(k1, g1c, 0.0), (CH, 16))
        d0_ref[sl, :] = jnp.where(k0, slot0, NB)
        d1_ref[sl, :] = jnp.where(k1, slot1, NB)
        sdrop = lax.slice(i0, (0, 0), (1, 1)) * CAP
        s0_ref[sl, :] = jnp.where(k0, slot0, sdrop)
        s1_ref[sl, :] = jnp.where(k1, slot1, sdrop)
        carry = carry + lax.slice(inc, (CH - 1, 0), (CH, E))


_router = pl.pallas_call(
    _router_body,
    out_shape=[
        jax.ShapeDtypeStruct((T, 1), jnp.int32),
        jax.ShapeDtypeStruct((T, 1), jnp.int32),
        jax.ShapeDtypeStruct((T, 16), jnp.float32),
        jax.ShapeDtypeStruct((T, 16), jnp.float32),
        jax.ShapeDtypeStruct((T, 1), jnp.int32),
        jax.ShapeDtypeStruct((T, 1), jnp.int32),
        jax.ShapeDtypeStruct((T, 1), jnp.int32),
        jax.ShapeDtypeStruct((T, 1), jnp.int32),
    ],
)


@functools.cache
def _sc_mesh():
    return plsc.VectorSubcoreMesh(
        core_axis_name="c", subcore_axis_name="s",
        num_cores=NC, num_subcores=NS)


_H = TPW // 2


@functools.cache
def _get_dispatch():
    @functools.partial(
        pl.kernel,
        out_type=jax.ShapeDtypeStruct((NBP, D), jnp.float32),
        mesh=_sc_mesh(),
        scratch_types=[
            pltpu.VMEM((_H,), jnp.int32),
            pltpu.VMEM((_H,), jnp.int32),
            pltpu.VMEM((_H,), jnp.int32),
            pltpu.VMEM((_H,), jnp.int32),
            pltpu.VMEM((_H, D), jnp.float32),
            pltpu.VMEM((_H, D), jnp.float32),
            pltpu.SemaphoreType.DMA,
            pltpu.SemaphoreType.DMA,
            pltpu.SemaphoreType.DMA,
            pltpu.SemaphoreType.DMA,
        ],
    )
    def _dispatch(x_hbm, d0_hbm, d1_hbm, buf_hbm,
                  i0a_v, i1a_v, i0b_v, i1b_v, ra_v, rb_v,
                  semA, semB, sem0, sem1):
        wid = lax.axis_index("s") * NC + lax.axis_index("c")
        base = wid * TPW
        pltpu.sync_copy(x_hbm.at[pl.ds(base, _H)], ra_v)
        pltpu.sync_copy(x_hbm.at[pl.ds(base + _H, _H)], rb_v)
        pltpu.sync_copy(d0_hbm.at[pl.ds(base, _H)], i0a_v)
        pltpu.sync_copy(d1_hbm.at[pl.ds(base, _H)], i1a_v)
        pltpu.sync_copy(d0_hbm.at[pl.ds(base + _H, _H)], i0b_v)
        pltpu.sync_copy(d1_hbm.at[pl.ds(base + _H, _H)], i1b_v)
        s0a = pltpu.async_copy(ra_v, buf_hbm.at[i0a_v], sem0)
        s1a = pltpu.async_copy(ra_v, buf_hbm.at[i1a_v], sem1)
        s0b = pltpu.async_copy(rb_v, buf_hbm.at[i0b_v], semA)
        s1b = pltpu.async_copy(rb_v, buf_hbm.at[i1b_v], semB)
        s0a.wait()
        s1a.wait()
        s0b.wait()
        s1b.wait()

    return _dispatch


def _ffn_body(b_ref, w1_ref, w2_ref, y_ref):
    f = pl.program_id(1)
    h = jnp.maximum(
        jnp.dot(b_ref[...], w1_ref[0], preferred_element_type=jnp.float32), 0.0)
    yc = jnp.dot(h, w2_ref[0], preferred_element_type=jnp.float32)

    @pl.when(f == 0)
    def _init():
        y_ref[...] = yc

    @pl.when(f > 0)
    def _acc():
        y_ref[...] += yc


_ffn = pl.pallas_call(
    _ffn_body,
    grid=(E, F // FC),
    in_specs=[
        pl.BlockSpec((CAP, D), lambda e, f: (e, 0)),
        pl.BlockSpec((1, D, FC), lambda e, f: (e, 0, f)),
        pl.BlockSpec((1, FC, D), lambda e, f: (e, f, 0)),
    ],
    out_specs=pl.BlockSpec((CAP, D), lambda e, f: (e, 0)),
    out_shape=jax.ShapeDtypeStruct((NB, D), jnp.float32),
    compiler_params=pltpu.CompilerParams(
        dimension_semantics=("parallel", "arbitrary")),
)


@functools.cache
def _get_combine():
    @functools.partial(
        pl.kernel,
        out_type=jax.ShapeDtypeStruct((T, D), jnp.float32),
        mesh=_sc_mesh(),
        scratch_types=[
            pltpu.VMEM((_H,), jnp.int32),
            pltpu.VMEM((_H,), jnp.int32),
            pltpu.VMEM((_H,), jnp.int32),
            pltpu.VMEM((_H,), jnp.int32),
            pltpu.VMEM((TPW, 16), jnp.float32),
            pltpu.VMEM((TPW, 16), jnp.float32),
            pltpu.VMEM((_H, D), jnp.float32),
            pltpu.VMEM((_H, D), jnp.float32),
            pltpu.VMEM((_H, D), jnp.float32),
            pltpu.VMEM((_H, D), jnp.float32),
            pltpu.SemaphoreType.DMA,
            pltpu.SemaphoreType.DMA,
            pltpu.SemaphoreType.DMA,
        ],
    )
    def _combine(y_hbm, s0_hbm, s1_hbm, g0_hbm, g1_hbm, o_hbm,
                 i0a_v, i1a_v, i0b_v, i1b_v, g0_v, g1_v,
                 r0a_v, r1a_v, r0b_v, r1b_v, semA, semB, semW):
        wid = lax.axis_index("s") * NC + lax.axis_index("c")
        base = wid * TPW
        pltpu.sync_copy(s0_hbm.at[pl.ds(base, _H)], i0a_v)
        pltpu.sync_copy(s1_hbm.at[pl.ds(base, _H)], i1a_v)
        pltpu.sync_copy(s0_hbm.at[pl.ds(base + _H, _H)], i0b_v)
        pltpu.sync_copy(s1_hbm.at[pl.ds(base + _H, _H)], i1b_v)
        ga0 = pltpu.async_copy(y_hbm.at[i0a_v], r0a_v, semA)
        ga1 = pltpu.async_copy(y_hbm.at[i1a_v], r1a_v, semA)
        gb0 = pltpu.async_copy(y_hbm.at[i0b_v], r0b_v, semB)
        gb1 = pltpu.async_copy(y_hbm.at[i1b_v], r1b_v, semB)
        pltpu.sync_copy(g0_hbm.at[pl.ds(base, TPW)], g0_v)
        pltpu.sync_copy(g1_hbm.at[pl.ds(base, TPW)], g1_v)

        def make_body(r0, r1, off):
            def body(i, carry):
                gv0 = g0_v[i + off, :]
                gv1 = g1_v[i + off, :]
                for j in range(D // 16):
                    sl = pl.ds(j * 16, 16)
                    r0[i, sl] = r0[i, sl] * gv0 + r1[i, sl] * gv1
                return carry
            return body

        ga0.wait()
        ga1.wait()
        lax.fori_loop(0, _H, make_body(r0a_v, r1a_v, 0), 0)
        wa = pltpu.async_copy(r0a_v, o_hbm.at[pl.ds(base, _H)], semW)
        gb0.wait()
        gb1.wait()
        lax.fori_loop(0, _H, make_body(r0b_v, r1b_v, _H), 0)
        wb = pltpu.async_copy(r0b_v, o_hbm.at[pl.ds(base + _H, _H)], semW)
        wa.wait()
        wb.wait()

    return _combine


def kernel(input_tensor, Wr, W1, W2):
    x = input_tensor
    i0, i1, g0, g1, d0, d1, s0, s1 = _router(x, Wr)
    buf = _get_dispatch()(x, d0.reshape(T), d1.reshape(T))
    y = _ffn(buf, W1, W2)
    out = _get_combine()(y, s0.reshape(T), s1.reshape(T), g0, g1)
    topi = jnp.concatenate([i0, i1], axis=1)
    return out, topi


# speedup vs baseline: 2.0233x; 1.0025x over previous
import functools

import jax
import jax.numpy as jnp
from jax import lax
from jax.experimental import pallas as pl
from jax.experimental.pallas import tpu as pltpu
from jax.experimental.pallas import tpu_sc as plsc

T = 2048
D = 768
E = 8
F = 3072
CAP = 640
NB = E * CAP
NBP = NB + 8
CH = 128
NC, NS = 2, 16
NW = NC * NS
TPW = T // NW
FC = 3072


def _router_body(x_ref, wr_ref,
                 i0_ref, i1_ref, g0_ref, g1_ref,
                 d0_ref, d1_ref, s0_ref, s1_ref):
    x = x_ref[...]
    wr = wr_ref[...]
    logits = jnp.dot(x, wr, preferred_element_type=jnp.float32)
    eidx = lax.broadcasted_iota(jnp.int32, (T, E), 1)
    m0 = jnp.max(logits, axis=1, keepdims=True)
    i0 = jnp.min(jnp.where(logits == m0, eidx, E), axis=1, keepdims=True)
    lmask = jnp.where(eidx == i0, -jnp.inf, logits)
    m1 = jnp.max(lmask, axis=1, keepdims=True)
    i1 = jnp.min(jnp.where(lmask == m1, eidx, E), axis=1, keepdims=True)
    g0 = 1.0 / (1.0 + jnp.exp(m1 - m0))
    g1 = 1.0 - g0

    onehot = ((eidx == i0) | (eidx == i1)).astype(jnp.float32)
    r = lax.broadcasted_iota(jnp.int32, (CH, CH), 0)
    c = lax.broadcasted_iota(jnp.int32, (CH, CH), 1)
    tril = (r >= c).astype(jnp.float32)
    eidx_c = lax.broadcasted_iota(jnp.int32, (CH, E), 1)
    carry = jnp.zeros((1, E), jnp.float32)
    for cb in range(T // CH):
        lo = cb * CH
        mc = lax.slice(onehot, (lo, 0), (lo + CH, E))
        inc = jnp.dot(tril, mc, preferred_element_type=jnp.float32)
        excl = inc - mc + carry
        i0c = lax.slice(i0, (lo, 0), (lo + CH, 1))
        i1c = lax.slice(i1, (lo, 0), (lo + CH, 1))
        g0c = lax.slice(g0, (lo, 0), (lo + CH, 1))
        g1c = lax.slice(g1, (lo, 0), (lo + CH, 1))
        p0 = jnp.sum(jnp.where(eidx_c == i0c, excl, 0.0), axis=1,
                     keepdims=True).astype(jnp.int32)
        p1 = jnp.sum(jnp.where(eidx_c == i1c, excl, 0.0), axis=1,
                     keepdims=True).astype(jnp.int32)
        k0 = p0 < CAP
        k1 = p1 < CAP
        slot0 = i0c * CAP + p0
        slot1 = i1c * CAP + p1
        sl = pl.ds(lo, CH)
        i0_ref[sl, :] = i0c
        i1_ref[sl, :] = i1c
        g0_ref[sl, :] = jnp.broadcast_to(jnp.where(k0, g0c, 0.0), (CH, 16))
        g1_ref[sl, :] = jnp.broadcast_to(jnp.where(k1, g1c, 0.0), (CH, 16))
        d0_ref[sl, :] = jnp.where(k0, slot0, NB)
        d1_ref[sl, :] = jnp.where(k1, slot1, NB)
        sdrop = lax.slice(i0, (0, 0), (1, 1)) * CAP
        s0_ref[sl, :] = jnp.where(k0, slot0, sdrop)
        s1_ref[sl, :] = jnp.where(k1, slot1, sdrop)
        carry = carry + lax.slice(inc, (CH - 1, 0), (CH, E))


_router = pl.pallas_call(
    _router_body,
    out_shape=[
        jax.ShapeDtypeStruct((T, 1), jnp.int32),
        jax.ShapeDtypeStruct((T, 1), jnp.int32),
        jax.ShapeDtypeStruct((T, 16), jnp.float32),
        jax.ShapeDtypeStruct((T, 16), jnp.float32),
        jax.ShapeDtypeStruct((T, 1), jnp.int32),
        jax.ShapeDtypeStruct((T, 1), jnp.int32),
        jax.ShapeDtypeStruct((T, 1), jnp.int32),
        jax.ShapeDtypeStruct((T, 1), jnp.int32),
    ],
)


@functools.cache
def _sc_mesh():
    return plsc.VectorSubcoreMesh(
        core_axis_name="c", subcore_axis_name="s",
        num_cores=NC, num_subcores=NS)


_H = TPW // 2


@functools.cache
def _get_dispatch():
    @functools.partial(
        pl.kernel,
        out_type=jax.ShapeDtypeStruct((NBP, D), jnp.float32),
        mesh=_sc_mesh(),
        scratch_types=[
            pltpu.VMEM((_H,), jnp.int32),
            pltpu.VMEM((_H,), jnp.int32),
            pltpu.VMEM((_H,), jnp.int32),
            pltpu.VMEM((_H,), jnp.int32),
            pltpu.VMEM((_H, D), jnp.float32),
            pltpu.VMEM((_H, D), jnp.float32),
            pltpu.SemaphoreType.DMA,
            pltpu.SemaphoreType.DMA,
            pltpu.SemaphoreType.DMA,
            pltpu.SemaphoreType.DMA,
        ],
    )
    def _dispatch(x_hbm, d0_hbm, d1_hbm, buf_hbm,
                  i0a_v, i1a_v, i0b_v, i1b_v, ra_v, rb_v,
                  semA, semB, sem0, sem1):
        wid = lax.axis_index("s") * NC + lax.axis_index("c")
        base = wid * TPW
        pltpu.sync_copy(d0_hbm.at[pl.ds(base, _H)], i0a_v)
        pltpu.sync_copy(d1_hbm.at[pl.ds(base, _H)], i1a_v)
        pltpu.sync_copy(x_hbm.at[pl.ds(base, _H)], ra_v)
        s0a = pltpu.async_copy(ra_v, buf_hbm.at[i0a_v], sem0)
        s1a = pltpu.async_copy(ra_v, buf_hbm.at[i1a_v], sem1)
        pltpu.sync_copy(d0_hbm.at[pl.ds(base + _H, _H)], i0b_v)
        pltpu.sync_copy(d1_hbm.at[pl.ds(base + _H, _H)], i1b_v)
        pltpu.sync_copy(x_hbm.at[pl.ds(base + _H, _H)], rb_v)
        s0b = pltpu.async_copy(rb_v, buf_hbm.at[i0b_v], semA)
        s1b = pltpu.async_copy(rb_v, buf_hbm.at[i1b_v], semB)
        s0a.wait()
        s1a.wait()
        s0b.wait()
        s1b.wait()

    return _dispatch


def _ffn_body(b_ref, w1_ref, w2_ref, y_ref):
    f = pl.program_id(1)
    h = jnp.maximum(
        jnp.dot(b_ref[...], w1_ref[0], preferred_element_type=jnp.float32), 0.0)
    yc = jnp.dot(h, w2_ref[0], preferred_element_type=jnp.float32)

    @pl.when(f == 0)
    def _init():
        y_ref[...] = yc

    @pl.when(f > 0)
    def _acc():
        y_ref[...] += yc


_ffn = pl.pallas_call(
    _ffn_body,
    grid=(E, F // FC),
    in_specs=[
        pl.BlockSpec((CAP, D), lambda e, f: (e, 0)),
        pl.BlockSpec((1, D, FC), lambda e, f: (e, 0, f)),
        pl.BlockSpec((1, FC, D), lambda e, f: (e, f, 0)),
    ],
    out_specs=pl.BlockSpec((CAP, D), lambda e, f: (e, 0)),
    out_shape=jax.ShapeDtypeStruct((NB, D), jnp.float32),
    compiler_params=pltpu.CompilerParams(
        dimension_semantics=("parallel", "arbitrary")),
)


@functools.cache
def _get_combine():
    @functools.partial(
        pl.kernel,
        out_type=jax.ShapeDtypeStruct((T, D), jnp.float32),
        mesh=_sc_mesh(),
        scratch_types=[
            pltpu.VMEM((_H,), jnp.int32),
            pltpu.VMEM((_H,), jnp.int32),
            pltpu.VMEM((_H,), jnp.int32),
            pltpu.VMEM((_H,), jnp.int32),
            pltpu.VMEM((TPW, 16), jnp.float32),
            pltpu.VMEM((TPW, 16), jnp.float32),
            pltpu.VMEM((_H, D), jnp.float32),
            pltpu.VMEM((_H, D), jnp.float32),
            pltpu.VMEM((_H, D), jnp.float32),
            pltpu.VMEM((_H, D), jnp.float32),
            pltpu.SemaphoreType.DMA,
            pltpu.SemaphoreType.DMA,
            pltpu.SemaphoreType.DMA,
        ],
    )
    def _combine(y_hbm, s0_hbm, s1_hbm, g0_hbm, g1_hbm, o_hbm,
                 i0a_v, i1a_v, i0b_v, i1b_v, g0_v, g1_v,
                 r0a_v, r1a_v, r0b_v, r1b_v, semA, semB, semW):
        wid = lax.axis_index("s") * NC + lax.axis_index("c")
        base = wid * TPW
        pltpu.sync_copy(s0_hbm.at[pl.ds(base, _H)], i0a_v)
        pltpu.sync_copy(s1_hbm.at[pl.ds(base, _H)], i1a_v)
        pltpu.sync_copy(s0_hbm.at[pl.ds(base + _H, _H)], i0b_v)
        pltpu.sync_copy(s1_hbm.at[pl.ds(base + _H, _H)], i1b_v)
        ga0 = pltpu.async_copy(y_hbm.at[i0a_v], r0a_v, semA)
        ga1 = pltpu.async_copy(y_hbm.at[i1a_v], r1a_v, semA)
        gb0 = pltpu.async_copy(y_hbm.at[i0b_v], r0b_v, semB)
        gb1 = pltpu.async_copy(y_hbm.at[i1b_v], r1b_v, semB)
        pltpu.sync_copy(g0_hbm.at[pl.ds(base, TPW)], g0_v)
        pltpu.sync_copy(g1_hbm.at[pl.ds(base, TPW)], g1_v)

        def make_body(r0, r1, off):
            def body(i, carry):
                gv0 = g0_v[i + off, :]
                gv1 = g1_v[i + off, :]
                for j in range(D // 16):
                    sl = pl.ds(j * 16, 16)
                    r0[i, sl] = r0[i, sl] * gv0 + r1[i, sl] * gv1
                return carry
            return body

        ga0.wait()
        ga1.wait()
        lax.fori_loop(0, _H, make_body(r0a_v, r1a_v, 0), 0)
        wa = pltpu.async_copy(r0a_v, o_hbm.at[pl.ds(base, _H)], semW)
        gb0.wait()
        gb1.wait()
        lax.fori_loop(0, _H, make_body(r0b_v, r1b_v, _H), 0)
        wb = pltpu.async_copy(r0b_v, o_hbm.at[pl.ds(base + _H, _H)], semW)
        wa.wait()
        wb.wait()

    return _combine


def kernel(input_tensor, Wr, W1, W2):
    x = input_tensor
    i0, i1, g0, g1, d0, d1, s0, s1 = _router(x, Wr)
    buf = _get_dispatch()(x, d0.reshape(T), d1.reshape(T))
    y = _ffn(buf, W1, W2)
    out = _get_combine()(y, s0.reshape(T), s1.reshape(T), g0, g1)
    topi = jnp.concatenate([i0, i1], axis=1)
    return out, topi


# speedup vs baseline: 2.0837x; 1.0298x over previous
import functools

import jax
import jax.numpy as jnp
from jax import lax
from jax.experimental import pallas as pl
from jax.experimental.pallas import tpu as pltpu
from jax.experimental.pallas import tpu_sc as plsc

T = 2048
D = 768
E = 8
F = 3072
CAP = 640
NB = E * CAP
NBP = NB + 8
CH = 128
NC, NS = 2, 16
NW = NC * NS
TPW = T // NW
FC = 3072


def _router_body(x_ref, wr_ref,
                 i0_ref, i1_ref, g0_ref, g1_ref,
                 d0_ref, d1_ref, s0_ref, s1_ref):
    x = x_ref[...]
    wr = wr_ref[...]
    logits = jnp.dot(x, wr, preferred_element_type=jnp.float32)
    eidx = lax.broadcasted_iota(jnp.int32, (T, E), 1)
    m0 = jnp.max(logits, axis=1, keepdims=True)
    i0 = jnp.min(jnp.where(logits == m0, eidx, E), axis=1, keepdims=True)
    lmask = jnp.where(eidx == i0, -jnp.inf, logits)
    m1 = jnp.max(lmask, axis=1, keepdims=True)
    i1 = jnp.min(jnp.where(lmask == m1, eidx, E), axis=1, keepdims=True)
    g0 = 1.0 / (1.0 + jnp.exp(m1 - m0))
    g1 = 1.0 - g0

    onehot = ((eidx == i0) | (eidx == i1)).astype(jnp.float32)
    r = lax.broadcasted_iota(jnp.int32, (CH, CH), 0)
    c = lax.broadcasted_iota(jnp.int32, (CH, CH), 1)
    tril = (r >= c).astype(jnp.float32)
    eidx_c = lax.broadcasted_iota(jnp.int32, (CH, E), 1)
    carry = jnp.zeros((1, E), jnp.float32)
    for cb in range(T // CH):
        lo = cb * CH
        mc = lax.slice(onehot, (lo, 0), (lo + CH, E))
        inc = jnp.dot(tril, mc, preferred_element_type=jnp.float32)
        excl = inc - mc + carry
        i0c = lax.slice(i0, (lo, 0), (lo + CH, 1))
        i1c = lax.slice(i1, (lo, 0), (lo + CH, 1))
        g0c = lax.slice(g0, (lo, 0), (lo + CH, 1))
        g1c = lax.slice(g1, (lo, 0), (lo + CH, 1))
        p0 = jnp.sum(jnp.where(eidx_c == i0c, excl, 0.0), axis=1,
                     keepdims=True).astype(jnp.int32)
        p1 = jnp.sum(jnp.where(eidx_c == i1c, excl, 0.0), axis=1,
                     keepdims=True).astype(jnp.int32)
        k0 = p0 < CAP
        k1 = p1 < CAP
        slot0 = i0c * CAP + p0
        slot1 = i1c * CAP + p1
        sl = pl.ds(lo, CH)
        i0_ref[sl, :] = i0c
        i1_ref[sl, :] = i1c
        g0_ref[sl, :] = jnp.broadcast_to(jnp.where(k0, g0c, 0.0), (CH, 16))
        g1_ref[sl, :] = jnp.broadcast_to(jnp.where(k1, g1c, 0.0), (CH, 16))
        d0_ref[sl, :] = jnp.where(k0, slot0, NB)
        d1_ref[sl, :] = jnp.where(k1, slot1, NB)
        sdrop = lax.slice(i0, (0, 0), (1, 1)) * CAP
        s0_ref[sl, :] = jnp.where(k0, slot0, sdrop)
        s1_ref[sl, :] = jnp.where(k1, slot1, sdrop)
        carry = carry + lax.slice(inc, (CH - 1, 0), (CH, E))


_router = pl.pallas_call(
    _router_body,
    out_shape=[
        jax.ShapeDtypeStruct((T, 1), jnp.int32),
        jax.ShapeDtypeStruct((T, 1), jnp.int32),
        jax.ShapeDtypeStruct((T, 16), jnp.float32),
        jax.ShapeDtypeStruct((T, 16), jnp.float32),
        jax.ShapeDtypeStruct((T, 1), jnp.int32),
        jax.ShapeDtypeStruct((T, 1), jnp.int32),
        jax.ShapeDtypeStruct((T, 1), jnp.int32),
        jax.ShapeDtypeStruct((T, 1), jnp.int32),
    ],
)


@functools.cache
def _sc_mesh():
    return plsc.VectorSubcoreMesh(
        core_axis_name="c", subcore_axis_name="s",
        num_cores=NC, num_subcores=NS)


_H = TPW // 2


@functools.cache
def _get_dispatch():
    @functools.partial(
        pl.kernel,
        out_type=jax.ShapeDtypeStruct((NBP, D), jnp.float32),
        mesh=_sc_mesh(),
        scratch_types=[
            pltpu.VMEM((_H,), jnp.int32),
            pltpu.VMEM((_H,), jnp.int32),
            pltpu.VMEM((_H,), jnp.int32),
            pltpu.VMEM((_H,), jnp.int32),
            pltpu.VMEM((_H, D), jnp.float32),
            pltpu.VMEM((_H, D), jnp.float32),
            pltpu.SemaphoreType.DMA,
            pltpu.SemaphoreType.DMA,
            pltpu.SemaphoreType.DMA,
            pltpu.SemaphoreType.DMA,
        ],
    )
    def _dispatch(x_hbm, d0_hbm, d1_hbm, buf_hbm,
                  i0a_v, i1a_v, i0b_v, i1b_v, ra_v, rb_v,
                  semA, semB, sem0, sem1):
        wid = lax.axis_index("s") * NC + lax.axis_index("c")
        base = wid * TPW
        pltpu.sync_copy(d0_hbm.at[pl.ds(base, _H)], i0a_v)
        pltpu.sync_copy(d1_hbm.at[pl.ds(base, _H)], i1a_v)
        pltpu.sync_copy(x_hbm.at[pl.ds(base, _H)], ra_v)
        s0a = pltpu.async_copy(ra_v, buf_hbm.at[i0a_v], sem0)
        s1a = pltpu.async_copy(ra_v, buf_hbm.at[i1a_v], sem1)
        pltpu.sync_copy(d0_hbm.at[pl.ds(base + _H, _H)], i0b_v)
        pltpu.sync_copy(d1_hbm.at[pl.ds(base + _H, _H)], i1b_v)
        pltpu.sync_copy(x_hbm.at[pl.ds(base + _H, _H)], rb_v)
        s0b = pltpu.async_copy(rb_v, buf_hbm.at[i0b_v], semA)
        s1b = pltpu.async_copy(rb_v, buf_hbm.at[i1b_v], semB)
        s0a.wait()
        s1a.wait()
        s0b.wait()
        s1b.wait()

    return _dispatch


def _ffn_body(b_ref, w1_ref, w2_ref, y_ref):
    f = pl.program_id(1)
    h = jnp.maximum(
        jnp.dot(b_ref[...], w1_ref[0], preferred_element_type=jnp.float32), 0.0)
    yc = jnp.dot(h, w2_ref[0], preferred_element_type=jnp.float32)

    @pl.when(f == 0)
    def _init():
        y_ref[...] = yc

    @pl.when(f > 0)
    def _acc():
        y_ref[...] += yc


_ffn = pl.pallas_call(
    _ffn_body,
    grid=(E, F // FC),
    in_specs=[
        pl.BlockSpec((CAP, D), lambda e, f: (e, 0)),
        pl.BlockSpec((1, D, FC), lambda e, f: (e, 0, f)),
        pl.BlockSpec((1, FC, D), lambda e, f: (e, f, 0)),
    ],
    out_specs=pl.BlockSpec((CAP, D), lambda e, f: (e, 0)),
    out_shape=jax.ShapeDtypeStruct((NB, D), jnp.float32),
    compiler_params=pltpu.CompilerParams(
        dimension_semantics=("parallel", "arbitrary")),
)


@functools.cache
def _get_combine():
    @functools.partial(
        pl.kernel,
        out_type=jax.ShapeDtypeStruct((T, D), jnp.float32),
        mesh=_sc_mesh(),
        scratch_types=[
            pltpu.VMEM((_H,), jnp.int32),
            pltpu.VMEM((_H,), jnp.int32),
            pltpu.VMEM((_H,), jnp.int32),
            pltpu.VMEM((_H,), jnp.int32),
            pltpu.VMEM((TPW, 16), jnp.float32),
            pltpu.VMEM((TPW, 16), jnp.float32),
            pltpu.VMEM((_H, D), jnp.float32),
            pltpu.VMEM((_H, D), jnp.float32),
            pltpu.VMEM((_H, D), jnp.float32),
            pltpu.VMEM((_H, D), jnp.float32),
            pltpu.SemaphoreType.DMA,
            pltpu.SemaphoreType.DMA,
            pltpu.SemaphoreType.DMA,
        ],
    )
    def _combine(y_hbm, s0_hbm, s1_hbm, g0_hbm, g1_hbm, o_hbm,
                 i0a_v, i1a_v, i0b_v, i1b_v, g0_v, g1_v,
                 r0a_v, r1a_v, r0b_v, r1b_v, semA, semB, semW):
        wid = lax.axis_index("s") * NC + lax.axis_index("c")
        base = wid * TPW
        pltpu.sync_copy(s0_hbm.at[pl.ds(base, _H)], i0a_v)
        pltpu.sync_copy(s1_hbm.at[pl.ds(base, _H)], i1a_v)
        pltpu.sync_copy(s0_hbm.at[pl.ds(base + _H, _H)], i0b_v)
        pltpu.sync_copy(s1_hbm.at[pl.ds(base + _H, _H)], i1b_v)
        ga0 = pltpu.async_copy(y_hbm.at[i0a_v], r0a_v, semA)
        ga1 = pltpu.async_copy(y_hbm.at[i1a_v], r1a_v, semA)
        gb0 = pltpu.async_copy(y_hbm.at[i0b_v], r0b_v, semB)
        gb1 = pltpu.async_copy(y_hbm.at[i1b_v], r1b_v, semB)
        pltpu.sync_copy(g0_hbm.at[pl.ds(base, TPW)], g0_v)
        pltpu.sync_copy(g1_hbm.at[pl.ds(base, TPW)], g1_v)

        def run_half(r0, r1, off):
            @functools.partial(plsc.parallel_loop, 0, _H, unroll=2)
            def body(i):
                gv0 = g0_v[i + off, :]
                gv1 = g1_v[i + off, :]
                for j in range(D // 16):
                    sl = pl.ds(j * 16, 16)
                    r0[i, sl] = r0[i, sl] * gv0 + r1[i, sl] * gv1

        ga0.wait()
        ga1.wait()
        run_half(r0a_v, r1a_v, 0)
        wa = pltpu.async_copy(r0a_v, o_hbm.at[pl.ds(base, _H)], semW)
        gb0.wait()
        gb1.wait()
        run_half(r0b_v, r1b_v, _H)
        wb = pltpu.async_copy(r0b_v, o_hbm.at[pl.ds(base + _H, _H)], semW)
        wa.wait()
        wb.wait()

    return _combine


def kernel(input_tensor, Wr, W1, W2):
    x = input_tensor
    i0, i1, g0, g1, d0, d1, s0, s1 = _router(x, Wr)
    buf = _get_dispatch()(x, d0.reshape(T), d1.reshape(T))
    y = _ffn(buf, W1, W2)
    out = _get_combine()(y, s0.reshape(T), s1.reshape(T), g0, g1)
    topi = jnp.concatenate([i0, i1], axis=1)
    return out, topi
